# baseline (speedup 1.0000x reference)
import functools

import jax
import jax.numpy as jnp
from jax import lax
from jax.experimental import pallas as pl
from jax.experimental.pallas import tpu as pltpu
from jax.experimental.pallas import tpu_sc as plsc


_K = 5
_BETA = 3.0
_NSUB = 16


def _body(q_ref, xpt_t_ref, xp_t_ref, xpt_s_ref, xp_s_ref, out_ref):
    q = q_ref[0, 0]
    qq = jnp.sum(q * q, axis=1, keepdims=True)

    res = []
    for xpt_ref, xp_ref in ((xpt_t_ref, xp_t_ref), (xpt_s_ref, xp_s_ref)):
        xt = xpt_ref[0]
        xp = xp_ref[0]
        xx = jnp.sum(xt * xt, axis=0, keepdims=True)
        d2 = qq + xx - 2.0 * jnp.dot(q, xt, preferred_element_type=jnp.float32)
        d2 = jnp.maximum(d2, 0.0)
        w_mat = jnp.zeros_like(d2)
        norm = jnp.zeros_like(qq)
        rem = jnp.full_like(qq, float(_K))
        for k in range(_K):
            m = jnp.min(d2, axis=1, keepdims=True)
            sel = (d2 == m).astype(jnp.float32)
            if k == 0:
                cnt = jnp.sum(sel, axis=1, keepdims=True)
                take = jnp.minimum(cnt, rem)
                w = take / (m + 1e-8)
                w_mat = w_mat + sel * (w / cnt)
            else:
                take = jnp.minimum(rem, 1.0)
                w = take / (m + 1e-8)
                w_mat = w_mat + sel * w
            norm = norm + w
            if k < _K - 1:
                rem = rem - take
                d2 = d2 + sel * jnp.float32(1e30)
        p = jnp.dot(w_mat, xp, preferred_element_type=jnp.float32)
        g = q - p / norm
        udf = jnp.sqrt(jnp.sum((g + 1e-10) ** 2, axis=1, keepdims=True))
        res.append((udf, g))

    (udf_t, g_t), (udf_s, g_s) = res
    ue = jnp.abs(udf_t - udf_s)
    ge = jnp.sum(jnp.abs(g_s - g_t), axis=1, keepdims=True)
    tot = ue + ge
    out_ref[0, 0] = tot * jnp.exp(-tot * _BETA)


def _build_query(src, tgt, noise):
    b, n_tgt, _ = tgt.shape
    query = tgt[:, :, None, :] + noise
    query = query.reshape(b, n_tgt * noise.shape[2], 3)
    return jnp.concatenate([query, src], axis=1)


def _tc_contrib(query, src, tgt, interpret=False):
    b, nq, _ = query.shape
    n_tgt, n_src = tgt.shape[1], src.shape[1]

    qblk = 256
    assert nq % qblk == 0
    nb = nq // qblk

    def pad8(a):
        return jnp.pad(a, ((0, 0), (0, 0), (0, 5)))

    qp = pad8(query).reshape(b, nb, qblk, 8)
    xp_t = pad8(tgt)
    xp_s = pad8(src)
    xpt_t = xp_t.transpose(0, 2, 1)
    xpt_s = xp_s.transpose(0, 2, 1)

    r_t = n_tgt
    r_s = n_src

    contrib = pl.pallas_call(
        _body,
        grid=(b, nb),
        in_specs=[
            pl.BlockSpec((1, 1, qblk, 8), lambda bi, i: (bi, i, 0, 0)),
            pl.BlockSpec((1, 8, r_t), lambda bi, i: (bi, 0, 0)),
            pl.BlockSpec((1, r_t, 8), lambda bi, i: (bi, 0, 0)),
            pl.BlockSpec((1, 8, r_s), lambda bi, i: (bi, 0, 0)),
            pl.BlockSpec((1, r_s, 8), lambda bi, i: (bi, 0, 0)),
        ],
        out_specs=pl.BlockSpec((1, 1, qblk, 1), lambda bi, i: (bi, i, 0, 0)),
        out_shape=jax.ShapeDtypeStruct((b, nb, qblk, 1), jnp.float32),
        interpret=interpret,
    )(qp, xpt_t, xp_t, xpt_s, xp_s)

    return contrib


@functools.partial(jax.jit, static_argnames=("interpret",))
def _impl(src, tgt, noise, interpret=False):
    query = _build_query(src, tgt, noise)
    contrib = _tc_contrib(query, src, tgt, interpret=interpret)
    return jnp.sum(contrib) / query.shape[0] / query.shape[1]


def _tail_body(gtx_ref, gty_ref, gtz_ref, gsx_ref, gsy_ref, gsz_ref, o_ref):
    gtx, gty, gtz = gtx_ref[0], gty_ref[0], gtz_ref[0]
    gsx, gsy, gsz = gsx_ref[0], gsy_ref[0], gsz_ref[0]

    def nrm(ax, ay, az):
        bx, by, bz = ax + 1e-10, ay + 1e-10, az + 1e-10
        return jnp.sqrt(bx * bx + by * by + bz * bz)

    tot = (jnp.abs(nrm(gtx, gty, gtz) - nrm(gsx, gsy, gsz))
           + jnp.abs(gsx - gtx) + jnp.abs(gsy - gty) + jnp.abs(gsz - gtz))
    o_ref[0] = tot * jnp.exp(tot * (-_BETA))


def _loss_tail(gt, gs):
    n = gt[0].shape[0]
    assert n % 1024 == 0 or n % 512 == 0
    rows = 8 if n % 1024 == 0 else 4
    nb = n // (rows * 128)
    args = [a.reshape(nb, rows, 128) for a in (*gt, *gs)]
    spec = pl.BlockSpec((1, rows, 128), lambda i: (i, 0, 0))
    contrib = pl.pallas_call(
        _tail_body,
        grid=(nb,),
        in_specs=[spec] * 6,
        out_specs=spec,
        out_shape=jax.ShapeDtypeStruct((nb, rows, 128), jnp.float32),
    )(*args)
    return jnp.sum(contrib)


@functools.lru_cache(maxsize=None)
def _make_sc(b, nqp, r):
    mesh = plsc.VectorSubcoreMesh(core_axis_name="c", subcore_axis_name="s")
    ngrp = nqp // 16
    nchunk = r // 16

    @functools.partial(
        pl.kernel,
        mesh=mesh,
        out_type=tuple(jax.ShapeDtypeStruct((b * _NSUB * nqp,), jnp.float32)
                       for _ in range(6)),
        scratch_types=[pltpu.VMEM((nqp,), jnp.float32)] * 3
        + [pltpu.VMEM((r,), jnp.float32)] * 6
        + [pltpu.VMEM((nqp,), jnp.float32)] * 6
        + [pltpu.VMEM((16, 16), jnp.float32), pltpu.SemaphoreType.DMA],
    )
    def sc_knn(qx_h, qy_h, qz_h, tx_h, ty_h, tz_h, sx_h, sy_h, sz_h,
               gtx_h, gty_h, gtz_h, gsx_h, gsy_h, gsz_h,
               qxv, qyv, qzv, txv, tyv, tzv, sxv, syv, szv,
               gtxv, gtyv, gtzv, gsxv, gsyv, gszv, gbuf, gsem):
        c = lax.axis_index("c")
        sid = lax.axis_index("s")
        nq = _NSUB * nqp
        base = c * nq + sid * nqp
        pltpu.sync_copy(qx_h.at[pl.ds(base, nqp)], qxv)
        pltpu.sync_copy(qy_h.at[pl.ds(base, nqp)], qyv)
        pltpu.sync_copy(qz_h.at[pl.ds(base, nqp)], qzv)
        rbase = c * r
        pltpu.sync_copy(tx_h.at[pl.ds(rbase, r)], txv)
        pltpu.sync_copy(ty_h.at[pl.ds(rbase, r)], tyv)
        pltpu.sync_copy(tz_h.at[pl.ds(rbase, r)], tzv)
        pltpu.sync_copy(sx_h.at[pl.ds(rbase, r)], sxv)
        pltpu.sync_copy(sy_h.at[pl.ds(rbase, r)], syv)
        pltpu.sync_copy(sz_h.at[pl.ds(rbase, r)], szv)

        def _bf(a):
            t = a * 65537.0
            return t - (t - a)

        def group(g, _):
            qx = qxv[pl.ds(g * 16, 16)]
            qy = qyv[pl.ds(g * 16, 16)]
            qz = qzv[pl.ds(g * 16, 16)]
            qxr, qyr, qzr = _bf(qx), _bf(qy), _bf(qz)
            qq = qx * qx + qy * qy + qz * qz
            outs = []
            for rxv, ryv, rzv, rx_h, ry_h, rz_h in (
                    (txv, tyv, tzv, tx_h, ty_h, tz_h),
                    (sxv, syv, szv, sx_h, sy_h, sz_h)):

                def chunk(cb, carry, rxv=rxv, ryv=ryv, rzv=rzv):
                    ts = list(carry[:5])
                    ids = list(carry[5:])
                    cb16 = cb * 16
                    rx16 = rxv[pl.ds(cb16, 16)]
                    ry16 = ryv[pl.ds(cb16, 16)]
                    rz16 = rzv[pl.ds(cb16, 16)]
                    xx16 = rx16 * rx16 + ry16 * ry16 + rz16 * rz16
                    rxr16, ryr16, rzr16 = _bf(rx16), _bf(ry16), _bf(rz16)
                    for j in range(16):
                        dot = (qxr * rxr16[j] + qyr * ryr16[j]
                               + qzr * rzr16[j])
                        x = jnp.maximum(qq + xx16[j] - 2.0 * dot, 0.0)
                        xi = jnp.full((16,), cb16 + j, jnp.int32)
                        for t in range(5):
                            swap = x < ts[t]
                            ts[t], x = (jnp.where(swap, x, ts[t]),
                                        jnp.where(swap, ts[t], x))
                            ids[t], xi = (jnp.where(swap, xi, ids[t]),
                                          jnp.where(swap, ids[t], xi))
                    return tuple(ts) + tuple(ids)

                init = (jnp.full((16,), 1e30, jnp.float32),) * 5 + (
                    jnp.zeros((16,), jnp.int32),) * 5
                fin = lax.fori_loop(0, nchunk, chunk, init)
                ts, ids = fin[:5], fin[5:]
                copies = []
                for k in range(5):
                    gi = ids[k] + rbase
                    for cc, r_h in enumerate((rx_h, ry_h, rz_h)):
                        copies.append(pltpu.async_copy(
                            r_h.at[gi], gbuf.at[k * 3 + cc], gsem))
                for cp in copies:
                    cp.wait()
                ws = [1.0 / (t + 1e-8) for t in ts]
                norm = ws[0] + ws[1] + ws[2] + ws[3] + ws[4]
                px = sum(ws[k] * gbuf[k * 3 + 0] for k in range(5))
                py = sum(ws[k] * gbuf[k * 3 + 1] for k in range(5))
                pz = sum(ws[k] * gbuf[k * 3 + 2] for k in range(5))
                outs.append((qx - px / norm, qy - py / norm, qz - pz / norm))

            (gtx, gty, gtz), (gsx, gsy, gsz) = outs
            sl = pl.ds(g * 16, 16)
            gtxv[sl], gtyv[sl], gtzv[sl] = gtx, gty, gtz
            gsxv[sl], gsyv[sl], gszv[sl] = gsx, gsy, gsz
            return 0

        lax.fori_loop(0, ngrp, group, 0)
        osl = pl.ds(base, nqp)
        pltpu.sync_copy(gtxv, gtx_h.at[osl])
        pltpu.sync_copy(gtyv, gty_h.at[osl])
        pltpu.sync_copy(gtzv, gtz_h.at[osl])
        pltpu.sync_copy(gsxv, gsx_h.at[osl])
        pltpu.sync_copy(gsyv, gsy_h.at[osl])
        pltpu.sync_copy(gszv, gsz_h.at[osl])

    return sc_knn


def _sc_grads(query, src, tgt):
    b, nq, _ = query.shape
    assert nq % (_NSUB * 16) == 0
    nqp = nq // _NSUB

    flat = lambda a, i: a[..., i].reshape(-1)
    args = [flat(query, i) for i in range(3)]
    args += [flat(tgt, i) for i in range(3)]
    args += [flat(src, i) for i in range(3)]
    return _make_sc(b, nqp, tgt.shape[1])(*args)


@jax.jit
def _impl_sc(src, tgt, noise):
    query = _build_query(src, tgt, noise)
    gtx, gty, gtz, gsx, gsy, gsz = _sc_grads(query, src, tgt)
    total = _loss_tail((gtx, gty, gtz), (gsx, gsy, gsz))
    return total / query.shape[0] / query.shape[1]


_TC_SHARE = 33024


@jax.jit
def _impl_hybrid(src, tgt, noise):
    query = _build_query(src, tgt, noise)
    b, nq, _ = query.shape
    grads = _sc_grads(query[:, _TC_SHARE:], src, tgt)
    tc_part = _tc_contrib(query[:, :_TC_SHARE], src, tgt)
    sc_total = _loss_tail(tuple(grads[:3]), tuple(grads[3:]))
    return (jnp.sum(tc_part) + sc_total) / b / nq


def kernel(src, tgt, noise):
    return _impl_hybrid(src, tgt, noise)


# speedup vs baseline: 1.0142x; 1.0142x over previous
import functools

import jax
import jax.numpy as jnp
from jax import lax
from jax.experimental import pallas as pl
from jax.experimental.pallas import tpu as pltpu
from jax.experimental.pallas import tpu_sc as plsc


_K = 5
_BETA = 3.0
_NSUB = 16


def _body(q_ref, xpt_t_ref, xp_t_ref, xpt_s_ref, xp_s_ref, out_ref):
    q = q_ref[0, 0]
    qq = jnp.sum(q * q, axis=1, keepdims=True)

    res = []
    for xpt_ref, xp_ref in ((xpt_t_ref, xp_t_ref), (xpt_s_ref, xp_s_ref)):
        xt = xpt_ref[0]
        xp = xp_ref[0]
        xx = jnp.sum(xt * xt, axis=0, keepdims=True)
        d2 = qq + xx - 2.0 * jnp.dot(q, xt, preferred_element_type=jnp.float32)
        d2 = jnp.maximum(d2, 0.0)
        w_mat = jnp.zeros_like(d2)
        norm = jnp.zeros_like(qq)
        rem = jnp.full_like(qq, float(_K))
        for k in range(_K):
            m = jnp.min(d2, axis=1, keepdims=True)
            sel = (d2 == m).astype(jnp.float32)
            if k == 0:
                cnt = jnp.sum(sel, axis=1, keepdims=True)
                take = jnp.minimum(cnt, rem)
                w = take / (m + 1e-8)
                w_mat = w_mat + sel * (w / cnt)
            else:
                take = jnp.minimum(rem, 1.0)
                w = take / (m + 1e-8)
                w_mat = w_mat + sel * w
            norm = norm + w
            if k < _K - 1:
                rem = rem - take
                d2 = d2 + sel * jnp.float32(1e30)
        p = jnp.dot(w_mat, xp, preferred_element_type=jnp.float32)
        g = q - p / norm
        udf = jnp.sqrt(jnp.sum((g + 1e-10) ** 2, axis=1, keepdims=True))
        res.append((udf, g))

    (udf_t, g_t), (udf_s, g_s) = res
    ue = jnp.abs(udf_t - udf_s)
    ge = jnp.sum(jnp.abs(g_s - g_t), axis=1, keepdims=True)
    tot = ue + ge
    out_ref[0, 0] = tot * jnp.exp(-tot * _BETA)


def _build_query(src, tgt, noise):
    b, n_tgt, _ = tgt.shape
    query = tgt[:, :, None, :] + noise
    query = query.reshape(b, n_tgt * noise.shape[2], 3)
    return jnp.concatenate([query, src], axis=1)


def _tc_contrib(query, src, tgt, interpret=False):
    b, nq, _ = query.shape
    n_tgt, n_src = tgt.shape[1], src.shape[1]

    qblk = 256
    assert nq % qblk == 0
    nb = nq // qblk

    def pad8(a):
        return jnp.pad(a, ((0, 0), (0, 0), (0, 5)))

    qp = pad8(query).reshape(b, nb, qblk, 8)
    xp_t = pad8(tgt)
    xp_s = pad8(src)
    xpt_t = xp_t.transpose(0, 2, 1)
    xpt_s = xp_s.transpose(0, 2, 1)

    r_t = n_tgt
    r_s = n_src

    contrib = pl.pallas_call(
        _body,
        grid=(b, nb),
        in_specs=[
            pl.BlockSpec((1, 1, qblk, 8), lambda bi, i: (bi, i, 0, 0)),
            pl.BlockSpec((1, 8, r_t), lambda bi, i: (bi, 0, 0)),
            pl.BlockSpec((1, r_t, 8), lambda bi, i: (bi, 0, 0)),
            pl.BlockSpec((1, 8, r_s), lambda bi, i: (bi, 0, 0)),
            pl.BlockSpec((1, r_s, 8), lambda bi, i: (bi, 0, 0)),
        ],
        out_specs=pl.BlockSpec((1, 1, qblk, 1), lambda bi, i: (bi, i, 0, 0)),
        out_shape=jax.ShapeDtypeStruct((b, nb, qblk, 1), jnp.float32),
        interpret=interpret,
    )(qp, xpt_t, xp_t, xpt_s, xp_s)

    return contrib


@functools.partial(jax.jit, static_argnames=("interpret",))
def _impl(src, tgt, noise, interpret=False):
    query = _build_query(src, tgt, noise)
    contrib = _tc_contrib(query, src, tgt, interpret=interpret)
    return jnp.sum(contrib) / query.shape[0] / query.shape[1]


def _tail_body(gtx_ref, gty_ref, gtz_ref, gsx_ref, gsy_ref, gsz_ref, o_ref):
    gtx, gty, gtz = gtx_ref[0], gty_ref[0], gtz_ref[0]
    gsx, gsy, gsz = gsx_ref[0], gsy_ref[0], gsz_ref[0]

    def nrm(ax, ay, az):
        bx, by, bz = ax + 1e-10, ay + 1e-10, az + 1e-10
        return jnp.sqrt(bx * bx + by * by + bz * bz)

    tot = (jnp.abs(nrm(gtx, gty, gtz) - nrm(gsx, gsy, gsz))
           + jnp.abs(gsx - gtx) + jnp.abs(gsy - gty) + jnp.abs(gsz - gtz))
    o_ref[0] = tot * jnp.exp(tot * (-_BETA))


def _loss_tail(gt, gs):
    n = gt[0].shape[0]
    assert n % 1024 == 0 or n % 512 == 0
    rows = 8 if n % 1024 == 0 else 4
    nb = n // (rows * 128)
    args = [a.reshape(nb, rows, 128) for a in (*gt, *gs)]
    spec = pl.BlockSpec((1, rows, 128), lambda i: (i, 0, 0))
    contrib = pl.pallas_call(
        _tail_body,
        grid=(nb,),
        in_specs=[spec] * 6,
        out_specs=spec,
        out_shape=jax.ShapeDtypeStruct((nb, rows, 128), jnp.float32),
    )(*args)
    return jnp.sum(contrib)


@functools.lru_cache(maxsize=None)
def _make_sc(b, nqp, r):
    mesh = plsc.VectorSubcoreMesh(core_axis_name="c", subcore_axis_name="s")
    ngrp = nqp // 16
    nchunk = r // 16

    @functools.partial(
        pl.kernel,
        mesh=mesh,
        out_type=tuple(jax.ShapeDtypeStruct((b * _NSUB * nqp,), jnp.float32)
                       for _ in range(6)),
        scratch_types=[pltpu.VMEM((nqp,), jnp.float32)] * 3
        + [pltpu.VMEM((r,), jnp.float32)] * 6
        + [pltpu.VMEM((nqp,), jnp.float32)] * 6
        + [pltpu.VMEM((r,), jnp.float32)] * 8
        + [pltpu.VMEM((16, 16), jnp.float32), pltpu.SemaphoreType.DMA],
    )
    def sc_knn(qx_h, qy_h, qz_h, tx_h, ty_h, tz_h, sx_h, sy_h, sz_h,
               gtx_h, gty_h, gtz_h, gsx_h, gsy_h, gsz_h,
               qxv, qyv, qzv, txv, tyv, tzv, sxv, syv, szv,
               gtxv, gtyv, gtzv, gsxv, gsyv, gszv,
               txr, tyr, tzr, sxr, syr, szr, txx, sxx, gbuf, gsem):
        c = lax.axis_index("c")
        sid = lax.axis_index("s")
        nq = _NSUB * nqp
        base = c * nq + sid * nqp
        pltpu.sync_copy(qx_h.at[pl.ds(base, nqp)], qxv)
        pltpu.sync_copy(qy_h.at[pl.ds(base, nqp)], qyv)
        pltpu.sync_copy(qz_h.at[pl.ds(base, nqp)], qzv)
        rbase = c * r
        pltpu.sync_copy(tx_h.at[pl.ds(rbase, r)], txv)
        pltpu.sync_copy(ty_h.at[pl.ds(rbase, r)], tyv)
        pltpu.sync_copy(tz_h.at[pl.ds(rbase, r)], tzv)
        pltpu.sync_copy(sx_h.at[pl.ds(rbase, r)], sxv)
        pltpu.sync_copy(sy_h.at[pl.ds(rbase, r)], syv)
        pltpu.sync_copy(sz_h.at[pl.ds(rbase, r)], szv)

        def _bf(a):
            t = a * 65537.0
            return t - (t - a)

        def prep(i, _):
            sl = pl.ds(i * 16, 16)
            tx, ty, tz = txv[sl], tyv[sl], tzv[sl]
            sx, sy, sz = sxv[sl], syv[sl], szv[sl]
            txr[sl], tyr[sl], tzr[sl] = _bf(tx), _bf(ty), _bf(tz)
            sxr[sl], syr[sl], szr[sl] = _bf(sx), _bf(sy), _bf(sz)
            txx[sl] = tx * tx + ty * ty + tz * tz
            sxx[sl] = sx * sx + sy * sy + sz * sz
            return 0

        lax.fori_loop(0, r // 16, prep, 0)

        def group(g, _):
            qx = qxv[pl.ds(g * 16, 16)]
            qy = qyv[pl.ds(g * 16, 16)]
            qz = qzv[pl.ds(g * 16, 16)]
            qxr, qyr, qzr = _bf(qx), _bf(qy), _bf(qz)
            qq = qx * qx + qy * qy + qz * qz
            outs = []
            for rxv, ryv, rzv, xxv, rx_h, ry_h, rz_h in (
                    (txr, tyr, tzr, txx, tx_h, ty_h, tz_h),
                    (sxr, syr, szr, sxx, sx_h, sy_h, sz_h)):

                def chunk(cb, carry, rxv=rxv, ryv=ryv, rzv=rzv, xxv=xxv):
                    ts = list(carry[:5])
                    ids = list(carry[5:])
                    cb16 = cb * 16
                    rxr16 = rxv[pl.ds(cb16, 16)]
                    ryr16 = ryv[pl.ds(cb16, 16)]
                    rzr16 = rzv[pl.ds(cb16, 16)]
                    xx16 = xxv[pl.ds(cb16, 16)]
                    for j in range(16):
                        dot = (qxr * rxr16[j] + qyr * ryr16[j]
                               + qzr * rzr16[j])
                        x = jnp.maximum(qq + xx16[j] - 2.0 * dot, 0.0)
                        xi = jnp.full((16,), cb16 + j, jnp.int32)
                        for t in range(5):
                            swap = x < ts[t]
                            ts[t], x = (jnp.where(swap, x, ts[t]),
                                        jnp.where(swap, ts[t], x))
                            ids[t], xi = (jnp.where(swap, xi, ids[t]),
                                          jnp.where(swap, ids[t], xi))
                    return tuple(ts) + tuple(ids)

                init = (jnp.full((16,), 1e30, jnp.float32),) * 5 + (
                    jnp.zeros((16,), jnp.int32),) * 5
                fin = lax.fori_loop(0, nchunk, chunk, init)
                ts, ids = fin[:5], fin[5:]
                copies = []
                for k in range(5):
                    gi = ids[k] + rbase
                    for cc, r_h in enumerate((rx_h, ry_h, rz_h)):
                        copies.append(pltpu.async_copy(
                            r_h.at[gi], gbuf.at[k * 3 + cc], gsem))
                for cp in copies:
                    cp.wait()
                ws = [1.0 / (t + 1e-8) for t in ts]
                norm = ws[0] + ws[1] + ws[2] + ws[3] + ws[4]
                px = sum(ws[k] * gbuf[k * 3 + 0] for k in range(5))
                py = sum(ws[k] * gbuf[k * 3 + 1] for k in range(5))
                pz = sum(ws[k] * gbuf[k * 3 + 2] for k in range(5))
                outs.append((qx - px / norm, qy - py / norm, qz - pz / norm))

            (gtx, gty, gtz), (gsx, gsy, gsz) = outs
            sl = pl.ds(g * 16, 16)
            gtxv[sl], gtyv[sl], gtzv[sl] = gtx, gty, gtz
            gsxv[sl], gsyv[sl], gszv[sl] = gsx, gsy, gsz
            return 0

        lax.fori_loop(0, ngrp, group, 0)
        osl = pl.ds(base, nqp)
        pltpu.sync_copy(gtxv, gtx_h.at[osl])
        pltpu.sync_copy(gtyv, gty_h.at[osl])
        pltpu.sync_copy(gtzv, gtz_h.at[osl])
        pltpu.sync_copy(gsxv, gsx_h.at[osl])
        pltpu.sync_copy(gsyv, gsy_h.at[osl])
        pltpu.sync_copy(gszv, gsz_h.at[osl])

    return sc_knn


def _sc_grads(query, src, tgt):
    b, nq, _ = query.shape
    assert nq % (_NSUB * 16) == 0
    nqp = nq // _NSUB

    flat = lambda a, i: a[..., i].reshape(-1)
    args = [flat(query, i) for i in range(3)]
    args += [flat(tgt, i) for i in range(3)]
    args += [flat(src, i) for i in range(3)]
    return _make_sc(b, nqp, tgt.shape[1])(*args)


@jax.jit
def _impl_sc(src, tgt, noise):
    query = _build_query(src, tgt, noise)
    gtx, gty, gtz, gsx, gsy, gsz = _sc_grads(query, src, tgt)
    total = _loss_tail((gtx, gty, gtz), (gsx, gsy, gsz))
    return total / query.shape[0] / query.shape[1]


_TC_SHARE = 32768


@jax.jit
def _impl_hybrid(src, tgt, noise):
    query = _build_query(src, tgt, noise)
    b, nq, _ = query.shape
    grads = _sc_grads(query[:, _TC_SHARE:], src, tgt)
    tc_part = _tc_contrib(query[:, :_TC_SHARE], src, tgt)
    sc_total = _loss_tail(tuple(grads[:3]), tuple(grads[3:]))
    return (jnp.sum(tc_part) + sc_total) / b / nq


def kernel(src, tgt, noise):
    return _impl_hybrid(src, tgt, noise)
